# Initial kernel scaffold; baseline (speedup 1.0000x reference)
import jax
import jax.numpy as jnp
from jax.experimental import pallas as pl


def kernel(x, level, grid0, grid1):
    raise NotImplementedError("write your pallas kernel here")


# speedup vs baseline: 1.4487x; 1.4487x over previous
import functools
import math

import jax
import jax.numpy as jnp
from jax import lax
from jax.experimental import pallas as pl
from jax.experimental.pallas import tpu as pltpu
from jax.experimental.pallas import tpu_sc as plsc

_N_LEVELS = 8
_PLANE = 512
_C = 16
_B = 131072

_NC, _NS = 2, 16
_NW = _NC * _NS
_BPW = _B // _NW
_CN = 64
_NCHUNK = _BPW // _CN

_UCOL = (1, 0, 0)
_VCOL = (2, 2, 1)


def _downsample(g):
    P, R, _, C = g.shape
    R2 = R // 2
    g6 = g.reshape(P, R2, 2, R2, 2, C)
    BR = min(8, R2)

    def body(g_ref, o_ref):
        o_ref[...] = 0.25 * (
            g_ref[:, :, 0, :, 0, :] + g_ref[:, :, 0, :, 1, :]
            + g_ref[:, :, 1, :, 0, :] + g_ref[:, :, 1, :, 1, :])

    return pl.pallas_call(
        body,
        grid=(P, R2 // BR),
        in_specs=[pl.BlockSpec((1, BR, 2, R2, 2, C),
                               lambda i, j: (i, j, 0, 0, 0, 0))],
        out_specs=pl.BlockSpec((1, BR, R2, C), lambda i, j: (i, j, 0, 0)),
        out_shape=jax.ShapeDtypeStruct((P, R2, R2, C), jnp.float32),
    )(g6)


def _static_offsets():
    offs = {}
    cur = 0
    for s in range(2):
        base_w = _PLANE * (1 << s)
        for p in range(3):
            for l in range(_N_LEVELS):
                w = base_w >> l
                offs[(s, p, l)] = cur
                cur += w * w
    return offs, cur


_OFFS, _T_ROWS = _static_offsets()


def _floor16(x):
    t = x.astype(jnp.int32).astype(jnp.float32)
    return jnp.where(x < t, t - 1.0, t)


def _sc_gather_interp(xin, table):
    mesh = plsc.VectorSubcoreMesh(core_axis_name="c", subcore_axis_name="s")

    @functools.partial(
        pl.kernel,
        mesh=mesh,
        out_type=jax.ShapeDtypeStruct((6, _B, _C), jnp.float32),
        scratch_types=[
            pltpu.VMEM((4, _CN), jnp.float32),
        ] + [pltpu.VMEM((_CN,), jnp.int32) for _ in range(8)] + [
            pltpu.VMEM((8 * _CN,), jnp.float32),
            pltpu.VMEM((8, _CN, 128), jnp.float32),
            pltpu.VMEM((_CN, _C), jnp.float32),
            pltpu.SemaphoreType.DMA,
        ],
    )
    def kern(xin_hbm, table_hbm, out_hbm, xin_v, i0, i1, i2, i3, i4, i5, i6,
             i7, w_v, rows_v, out_v, sem):
        idxs = (i0, i1, i2, i3, i4, i5, i6, i7)
        wid = lax.axis_index("s") * _NC + lax.axis_index("c")
        wbase = wid * _BPW

        def chunk_body(ci, _):
            base = wbase + ci * _CN
            for j in range(4):
                pltpu.sync_copy(xin_hbm.at[j, pl.ds(base, _CN)], xin_v.at[j])

            for s in range(2):
                bias = float(math.log2(_PLANE) + s)
                max_w = float(_PLANE * (1 << s))
                for p in range(3):
                    sp = s * 3 + p

                    def t_body(t, _, s=s, p=p, bias=bias, max_w=max_w):
                        off16 = t * 16
                        u = xin_v[_UCOL[p], pl.ds(off16, 16)]
                        v = xin_v[_VCOL[p], pl.ds(off16, 16)]
                        lv = xin_v[3, pl.ds(off16, 16)] + bias
                        lv = jnp.clip(lv, 0.0, float(_N_LEVELS - 1))
                        l0 = _floor16(lv)
                        fmip = lv - l0
                        l1 = jnp.minimum(l0 + 1.0, float(_N_LEVELS - 1))
                        for mi, (lm, wm) in enumerate(
                                ((l0, 1.0 - fmip), (l1, fmip))):
                            wf = jnp.full((16,), max_w, jnp.float32)
                            offf = jnp.full((16,), float(_OFFS[(s, p, 0)]),
                                            jnp.float32)
                            for l in range(1, _N_LEVELS):
                                sel = lm == float(l)
                                wf = jnp.where(sel, max_w / (1 << l), wf)
                                offf = jnp.where(
                                    sel, float(_OFFS[(s, p, l)]), offf)
                            xx = u * wf - 0.5
                            yy = v * wf - 0.5
                            x0f = _floor16(xx)
                            y0f = _floor16(yy)
                            fx = xx - x0f
                            fy = yy - y0f
                            x0 = jnp.clip(x0f, 0.0, wf - 1.0)
                            x1 = jnp.clip(x0f + 1.0, 0.0, wf - 1.0)
                            y0 = jnp.clip(y0f, 0.0, wf - 1.0)
                            y1 = jnp.clip(y0f + 1.0, 0.0, wf - 1.0)
                            k0 = 4 * mi
                            for k, (xc, yc, wc) in enumerate((
                                    (x0, y0, (1.0 - fx) * (1.0 - fy)),
                                    (x1, y0, fx * (1.0 - fy)),
                                    (x0, y1, (1.0 - fx) * fy),
                                    (x1, y1, fx * fy))):
                                idxs[k0 + k][pl.ds(off16, 16)] = (
                                    offf + yc * wf + xc).astype(jnp.int32)
                                w_v[pl.ds((k0 + k) * _CN + off16, 16)] = (
                                    wc * wm)
                        return 0

                    lax.fori_loop(0, _CN // 16, t_body, 0)

                    copies = [
                        pltpu.async_copy(table_hbm.at[idxs[k]],
                                         rows_v.at[k], sem)
                        for k in range(8)
                    ]
                    for c in copies:
                        c.wait()

                    def g_body(g, _):
                        gb = g * 16
                        w16s = [w_v[pl.ds(k * _CN + gb, 16)]
                                for k in range(8)]
                        for lane in range(16):
                            idxv = jnp.full((16,), lane, jnp.int32)
                            acc = jnp.zeros((16,), jnp.float32)
                            for k in range(8):
                                wsp = w16s[k].at[idxv].get(
                                    mode="promise_in_bounds")
                                acc = acc + wsp * rows_v[
                                    k, gb + lane, pl.ds(0, _C)]
                            out_v[gb + lane, :] = acc
                        return 0

                    lax.fori_loop(0, _CN // 16, g_body, 0)
                    pltpu.sync_copy(out_v, out_hbm.at[sp, pl.ds(base, _CN)])
            return 0

        lax.fori_loop(0, _NCHUNK, chunk_body, 0)

    return kern(xin, table)


def kernel(x, level, grid0, grid1):
    parts = []
    for g in (grid0, grid1):
        levels = [g]
        for _ in range(_N_LEVELS - 1):
            levels.append(_downsample(levels[-1]))
        for p in range(3):
            for lv in levels:
                parts.append(lv[p].reshape(-1, _C))
    table = jnp.concatenate(parts, axis=0)
    table = jnp.pad(table, ((0, 0), (0, 128 - _C)))

    xin = jnp.concatenate([x.T, level.T], axis=0)
    out6 = _sc_gather_interp(xin, table)
    return out6.transpose(1, 0, 2).reshape(_B, 6 * _C)
